# baseline (speedup 1.0000x reference)
import functools

import jax
import jax.numpy as jnp
from jax import lax
from jax.experimental import pallas as pl
from jax.experimental.pallas import tpu as pltpu
from jax.experimental.pallas import tpu_sc as plsc

E = 64
K = 8
NT = 32768
L = 16
NC, NS = 2, 16
NW = NC * NS

NT_SC = 8192
NT_TC = NT - NT_SC
TSUB = NT_SC // NW
TB = 4096


def _sc_router_body(logits_hbm, w_hbm, ids_hbm, hist_hbm, lg_v, w_v, ids_v, hist_v):
    wid = lax.axis_index("s") * NC + lax.axis_index("c")
    base = wid * TSUB

    pltpu.sync_copy(logits_hbm.at[pl.ds(base * E, TSUB * E)], lg_v)

    lane = lax.iota(jnp.int32, L)
    low8 = lane < 8
    ones = jnp.ones((L,), jnp.float32)
    zeros = jnp.zeros((L,), jnp.float32)
    group_ids = [lane + g * L for g in range(E // L)]

    for g in range(E // L):
        hist_v[pl.ds(g * L, L)] = zeros

    perm7 = jnp.maximum(7 - lane, 0)
    shift8 = jnp.maximum(lane - 8, 0)

    gather_dnums = lax.GatherDimensionNumbers(
        offset_dims=(), collapsed_slice_dims=(0,), start_index_map=(0,)
    )

    def take(x, idx):
        return lax.gather(
            x, idx[:, None], gather_dnums, (1,),
            mode=lax.GatherScatterMode.PROMISE_IN_BOUNDS,
        )

    def cmerge(a, b):
        av, ai = a
        bv, bi = b
        bvp, bip = take(bv, perm7), take(bi, perm7)
        c = av >= bvp
        return jnp.where(c, av, bvp), jnp.where(c, ai, bip)

    def one_token(t):
        o = t * E
        runs = []
        for g in range(E // L):
            v = lg_v[pl.ds(o + g * L, L)]
            runs.append(plsc.sort_key_val(v, group_ids[g], descending=True))
        m01v, m01i = cmerge(runs[0], runs[1])
        m23v, m23i = cmerge(runs[2], runs[3])
        cv = jnp.where(low8, m01v, take(m23v, shift8))
        ci = jnp.where(low8, m01i, take(m23i, shift8))
        fv, fi = plsc.sort_key_val(cv, ci, descending=True)
        w = jnp.exp(fv - jnp.max(fv))
        w = jnp.where(low8, w, 0.0)
        w = w / jnp.sum(w)
        out_idx = t * K + lane
        plsc.store_scatter(w_v, [out_idx], w, mask=low8)
        plsc.store_scatter(ids_v, [out_idx], fi, mask=low8)
        plsc.addupdate_scatter(hist_v, [fi], ones, mask=low8)

    UNROLL = 4
    def token_body(i, _):
        t0 = i * UNROLL
        for u in range(UNROLL):
            one_token(t0 + u)
        return _

    lax.fori_loop(0, TSUB // UNROLL, token_body, None)

    pltpu.sync_copy(w_v, w_hbm.at[pl.ds(base * K, TSUB * K)])
    pltpu.sync_copy(ids_v, ids_hbm.at[pl.ds(base * K, TSUB * K)])
    pltpu.sync_copy(hist_v, hist_hbm.at[pl.ds(wid * E, E)])


_sc_router = functools.partial(
    pl.kernel,
    mesh=plsc.VectorSubcoreMesh(
        core_axis_name="c", subcore_axis_name="s", num_cores=NC, num_subcores=NS
    ),
    out_type=(
        jax.ShapeDtypeStruct((NT_SC * K,), jnp.float32),
        jax.ShapeDtypeStruct((NT_SC * K,), jnp.int32),
        jax.ShapeDtypeStruct((NW * E,), jnp.float32),
    ),
    scratch_types=[
        pltpu.VMEM((TSUB * E,), jnp.float32),
        pltpu.VMEM((TSUB * K,), jnp.float32),
        pltpu.VMEM((TSUB * K,), jnp.int32),
        pltpu.VMEM((E,), jnp.float32),
    ],
    compiler_params=pltpu.CompilerParams(needs_layout_passes=False),
)(_sc_router_body)


def _tc_router_body(x_ref, xout_ref, w_ref, ids_ref, cnt_ref):
    i = pl.program_id(0)
    x = x_ref[...]
    xout_ref[...] = x

    @pl.when(i == 0)
    def _():
        cnt_ref[...] = jnp.zeros((E,), jnp.float32)

    @pl.when(i < NT_TC // TB)
    def _():
        iota_e = lax.broadcasted_iota(jnp.int32, (E, TB), 0)
        bits = lax.bitcast_convert_type(x, jnp.int32)
        lowsel = jnp.where(bits < 0, iota_e, 63 - iota_e)
        key = lax.bitcast_convert_type((bits & jnp.int32(~63)) | lowsel, jnp.float32)
        mkeys = []
        for k in range(K):
            mkey = jnp.max(key, axis=0, keepdims=True)
            key = jnp.where(key == mkey, -jnp.inf, key)
            mkeys.append(mkey)
        mk = jnp.concatenate(mkeys, axis=0)
        mkb = lax.bitcast_convert_type(mk, jnp.int32)
        idc = jnp.where(mkb < 0, mkb & 63, 63 - (mkb & 63))
        w = jnp.exp(mk - mk[0:1, :])
        w_ref[...] = w / jnp.sum(w, axis=0, keepdims=True)
        ids_ref[...] = idc
        sel = (key == -jnp.inf).astype(jnp.float32)
        cnt_ref[...] += jnp.sum(sel, axis=1)


_tc_router = pl.pallas_call(
    _tc_router_body,
    grid=(NT // TB,),
    in_specs=[pl.BlockSpec((E, TB), lambda i: (0, i))],
    out_specs=[
        pl.BlockSpec((E, TB), lambda i: (0, i)),
        pl.BlockSpec((K, TB), lambda i: (0, i)),
        pl.BlockSpec((K, TB), lambda i: (0, i)),
        pl.BlockSpec((E,), lambda i: (0,)),
    ],
    out_shape=(
        jax.ShapeDtypeStruct((E, NT), jnp.float32),
        jax.ShapeDtypeStruct((K, NT), jnp.float32),
        jax.ShapeDtypeStruct((K, NT), jnp.int32),
        jax.ShapeDtypeStruct((E,), jnp.float32),
    ),
)


def _hist_reduce_body(p_ref, t_ref, o_ref):
    o_ref[...] = jnp.sum(p_ref[...], axis=0) + t_ref[...]


def kernel(logits):
    xT_out, wT_tc, idsT_tc, cnt_tc = _tc_router(logits.T)
    w_sc, ids_sc, partials = _sc_router(logits[NT_TC:, :].reshape(-1))
    tokens_per_expert = pl.pallas_call(
        _hist_reduce_body,
        out_shape=jax.ShapeDtypeStruct((E,), jnp.float32),
    )(partials.reshape(NW, E), cnt_tc)
    wT = jnp.concatenate([wT_tc[:, :NT_TC], w_sc.reshape(NT_SC, K).T], axis=1)
    idsT = jnp.concatenate([idsT_tc[:, :NT_TC], ids_sc.reshape(NT_SC, K).T], axis=1)
    return (xT_out.T, wT.T, idsT.T, tokens_per_expert)


# speedup vs baseline: 1.2537x; 1.2537x over previous
import functools

import jax
import jax.numpy as jnp
from jax import lax
from jax.experimental import pallas as pl
from jax.experimental.pallas import tpu as pltpu
from jax.experimental.pallas import tpu_sc as plsc

E = 64
K = 8
NT = 32768
L = 16
NC, NS = 2, 16
NW = NC * NS

NT_SC = 4096
NT_TC = NT - NT_SC
TSUB = NT_SC // NW
TB = 4096


def _sc_router_body(logits_hbm, w_hbm, ids_hbm, hist_hbm, lg_v, w_v, ids_v, hist_v):
    wid = lax.axis_index("s") * NC + lax.axis_index("c")
    base = wid * TSUB

    pltpu.sync_copy(logits_hbm.at[pl.ds(base * E, TSUB * E)], lg_v)

    lane = lax.iota(jnp.int32, L)
    low8 = lane < 8
    ones = jnp.ones((L,), jnp.float32)
    zeros = jnp.zeros((L,), jnp.float32)
    group_ids = [lane + g * L for g in range(E // L)]

    for g in range(E // L):
        hist_v[pl.ds(g * L, L)] = zeros

    perm7 = jnp.maximum(7 - lane, 0)
    shift8 = jnp.maximum(lane - 8, 0)

    gather_dnums = lax.GatherDimensionNumbers(
        offset_dims=(), collapsed_slice_dims=(0,), start_index_map=(0,)
    )

    def take(x, idx):
        return lax.gather(
            x, idx[:, None], gather_dnums, (1,),
            mode=lax.GatherScatterMode.PROMISE_IN_BOUNDS,
        )

    def cmerge(a, b):
        av, ai = a
        bv, bi = b
        bvp, bip = take(bv, perm7), take(bi, perm7)
        c = av >= bvp
        return jnp.where(c, av, bvp), jnp.where(c, ai, bip)

    def one_token(t):
        o = t * E
        runs = []
        for g in range(E // L):
            v = lg_v[pl.ds(o + g * L, L)]
            runs.append(plsc.sort_key_val(v, group_ids[g], descending=True))
        m01v, m01i = cmerge(runs[0], runs[1])
        m23v, m23i = cmerge(runs[2], runs[3])
        cv = jnp.where(low8, m01v, take(m23v, shift8))
        ci = jnp.where(low8, m01i, take(m23i, shift8))
        fv, fi = plsc.sort_key_val(cv, ci, descending=True)
        w = jnp.exp(fv - jnp.max(fv))
        w = jnp.where(low8, w, 0.0)
        w = w / jnp.sum(w)
        out_idx = t * K + lane
        plsc.store_scatter(w_v, [out_idx], w, mask=low8)
        plsc.store_scatter(ids_v, [out_idx], fi, mask=low8)
        plsc.addupdate_scatter(hist_v, [fi], ones, mask=low8)

    UNROLL = 4
    def token_body(i, _):
        t0 = i * UNROLL
        for u in range(UNROLL):
            one_token(t0 + u)
        return _

    lax.fori_loop(0, TSUB // UNROLL, token_body, None)

    pltpu.sync_copy(w_v, w_hbm.at[pl.ds(base * K, TSUB * K)])
    pltpu.sync_copy(ids_v, ids_hbm.at[pl.ds(base * K, TSUB * K)])
    pltpu.sync_copy(hist_v, hist_hbm.at[pl.ds(wid * E, E)])


_sc_router = functools.partial(
    pl.kernel,
    mesh=plsc.VectorSubcoreMesh(
        core_axis_name="c", subcore_axis_name="s", num_cores=NC, num_subcores=NS
    ),
    out_type=(
        jax.ShapeDtypeStruct((NT_SC * K,), jnp.float32),
        jax.ShapeDtypeStruct((NT_SC * K,), jnp.int32),
        jax.ShapeDtypeStruct((NW * E,), jnp.float32),
    ),
    scratch_types=[
        pltpu.VMEM((TSUB * E,), jnp.float32),
        pltpu.VMEM((TSUB * K,), jnp.float32),
        pltpu.VMEM((TSUB * K,), jnp.int32),
        pltpu.VMEM((E,), jnp.float32),
    ],
    compiler_params=pltpu.CompilerParams(needs_layout_passes=False),
)(_sc_router_body)


def _tc_router_body(x_ref, xout_ref, w_ref, ids_ref, cnt_ref):
    i = pl.program_id(0)
    x = x_ref[...]
    xout_ref[...] = x

    @pl.when(i == 0)
    def _():
        cnt_ref[...] = jnp.zeros((E,), jnp.float32)

    @pl.when(i < NT_TC // TB)
    def _():
        iota_e = lax.broadcasted_iota(jnp.int32, (E, TB), 0)
        bits = lax.bitcast_convert_type(x, jnp.int32)
        lowsel = jnp.where(bits < 0, iota_e, 63 - iota_e)
        key = lax.bitcast_convert_type((bits & jnp.int32(~63)) | lowsel, jnp.float32)
        mkeys = []
        for k in range(K):
            mkey = jnp.max(key, axis=0, keepdims=True)
            key = jnp.where(key == mkey, -jnp.inf, key)
            mkeys.append(mkey)
        mk = jnp.concatenate(mkeys, axis=0)
        mkb = lax.bitcast_convert_type(mk, jnp.int32)
        idc = jnp.where(mkb < 0, mkb & 63, 63 - (mkb & 63))
        w = jnp.exp(mk - mk[0:1, :])
        w_ref[...] = w / jnp.sum(w, axis=0, keepdims=True)
        ids_ref[...] = idc
        sel = (key == -jnp.inf).astype(jnp.float32)
        cnt_ref[...] += jnp.sum(sel, axis=1)


_tc_router = pl.pallas_call(
    _tc_router_body,
    grid=(NT // TB,),
    in_specs=[pl.BlockSpec((E, TB), lambda i: (0, i))],
    out_specs=[
        pl.BlockSpec((E, TB), lambda i: (0, i)),
        pl.BlockSpec((K, TB), lambda i: (0, i)),
        pl.BlockSpec((K, TB), lambda i: (0, i)),
        pl.BlockSpec((E,), lambda i: (0,)),
    ],
    out_shape=(
        jax.ShapeDtypeStruct((E, NT), jnp.float32),
        jax.ShapeDtypeStruct((K, NT), jnp.float32),
        jax.ShapeDtypeStruct((K, NT), jnp.int32),
        jax.ShapeDtypeStruct((E,), jnp.float32),
    ),
)


def _hist_reduce_body(p_ref, t_ref, o_ref):
    o_ref[...] = jnp.sum(p_ref[...], axis=0) + t_ref[...]


def kernel(logits):
    xT_out, wT_tc, idsT_tc, cnt_tc = _tc_router(logits.T)
    w_sc, ids_sc, partials = _sc_router(logits[NT_TC:, :].reshape(-1))
    tokens_per_expert = pl.pallas_call(
        _hist_reduce_body,
        out_shape=jax.ShapeDtypeStruct((E,), jnp.float32),
    )(partials.reshape(NW, E), cnt_tc)
    wT = jnp.concatenate([wT_tc[:, :NT_TC], w_sc.reshape(NT_SC, K).T], axis=1)
    idsT = jnp.concatenate([idsT_tc[:, :NT_TC], ids_sc.reshape(NT_SC, K).T], axis=1)
    return (xT_out.T, wT.T, idsT.T, tokens_per_expert)


# speedup vs baseline: 1.3270x; 1.0584x over previous
import functools

import jax
import jax.numpy as jnp
from jax import lax
from jax.experimental import pallas as pl
from jax.experimental.pallas import tpu as pltpu
from jax.experimental.pallas import tpu_sc as plsc

E = 64
K = 8
NT = 32768
L = 16
NC, NS = 2, 16
NW = NC * NS

NT_SC = 4096
NT_TC = NT - NT_SC
TSUB = NT_SC // NW
TB = 4096


def _sc_router_body(logits_hbm, w_hbm, ids_hbm, hist_hbm, lg_v, w_v, ids_v, hist_v):
    wid = lax.axis_index("s") * NC + lax.axis_index("c")
    base = wid * TSUB

    pltpu.sync_copy(logits_hbm.at[pl.ds(base * E, TSUB * E)], lg_v)

    lane = lax.iota(jnp.int32, L)
    low8 = lane < 8
    ones = jnp.ones((L,), jnp.float32)
    zeros = jnp.zeros((L,), jnp.float32)
    group_ids = [lane + g * L for g in range(E // L)]

    for g in range(E // L):
        hist_v[pl.ds(g * L, L)] = zeros

    perm7 = jnp.maximum(7 - lane, 0)
    shift8 = jnp.maximum(lane - 8, 0)

    gather_dnums = lax.GatherDimensionNumbers(
        offset_dims=(), collapsed_slice_dims=(0,), start_index_map=(0,)
    )

    def take(x, idx):
        return lax.gather(
            x, idx[:, None], gather_dnums, (1,),
            mode=lax.GatherScatterMode.PROMISE_IN_BOUNDS,
        )

    def cmerge(a, b):
        av, ai = a
        bv, bi = b
        bvp, bip = take(bv, perm7), take(bi, perm7)
        c = av >= bvp
        return jnp.where(c, av, bvp), jnp.where(c, ai, bip)

    def one_token(t):
        o = t * E
        runs = []
        for g in range(E // L):
            v = lg_v[pl.ds(o + g * L, L)]
            runs.append(plsc.sort_key_val(v, group_ids[g], descending=True))
        m01v, m01i = cmerge(runs[0], runs[1])
        m23v, m23i = cmerge(runs[2], runs[3])
        cv = jnp.where(low8, m01v, take(m23v, shift8))
        ci = jnp.where(low8, m01i, take(m23i, shift8))
        fv, fi = plsc.sort_key_val(cv, ci, descending=True)
        w = jnp.exp(fv - jnp.max(fv))
        w = jnp.where(low8, w, 0.0)
        w = w / jnp.sum(w)
        out_idx = t * K + lane
        plsc.store_scatter(w_v, [out_idx], w, mask=low8)
        plsc.store_scatter(ids_v, [out_idx], fi, mask=low8)
        plsc.addupdate_scatter(hist_v, [fi], ones, mask=low8)

    UNROLL = 4
    def token_body(i, _):
        t0 = i * UNROLL
        for u in range(UNROLL):
            one_token(t0 + u)
        return _

    lax.fori_loop(0, TSUB // UNROLL, token_body, None)

    pltpu.sync_copy(w_v, w_hbm.at[pl.ds(base * K, TSUB * K)])
    pltpu.sync_copy(ids_v, ids_hbm.at[pl.ds(base * K, TSUB * K)])
    pltpu.sync_copy(hist_v, hist_hbm.at[pl.ds(wid * E, E)])


_sc_router = functools.partial(
    pl.kernel,
    mesh=plsc.VectorSubcoreMesh(
        core_axis_name="c", subcore_axis_name="s", num_cores=NC, num_subcores=NS
    ),
    out_type=(
        jax.ShapeDtypeStruct((NT_SC * K,), jnp.float32),
        jax.ShapeDtypeStruct((NT_SC * K,), jnp.int32),
        jax.ShapeDtypeStruct((NW * E,), jnp.float32),
    ),
    scratch_types=[
        pltpu.VMEM((TSUB * E,), jnp.float32),
        pltpu.VMEM((TSUB * K,), jnp.float32),
        pltpu.VMEM((TSUB * K,), jnp.int32),
        pltpu.VMEM((E,), jnp.float32),
    ],
    compiler_params=pltpu.CompilerParams(needs_layout_passes=False),
)(_sc_router_body)


def _tc_router_body(x_ref, xout_ref, w_ref, ids_ref, cnt_ref):
    i = pl.program_id(0)
    x = x_ref[...]
    xout_ref[...] = x

    @pl.when(i == 0)
    def _():
        cnt_ref[...] = jnp.zeros((E,), jnp.float32)

    @pl.when(i < NT_TC // TB)
    def _():
        iota_e = lax.broadcasted_iota(jnp.int32, (E, TB), 0)
        bits = lax.bitcast_convert_type(x, jnp.int32)
        lowsel = jnp.where(bits < 0, iota_e, 63 - iota_e)
        key = lax.bitcast_convert_type((bits & jnp.int32(~63)) | lowsel, jnp.float32)
        mkeys = []
        for k in range(K):
            mkey = jnp.max(key, axis=0, keepdims=True)
            key = jnp.where(key == mkey, -jnp.inf, key)
            mkeys.append(mkey)
        mk = jnp.concatenate(mkeys, axis=0)
        mkb = lax.bitcast_convert_type(mk, jnp.int32)
        idc = jnp.where(mkb < 0, mkb & 63, 63 - (mkb & 63))
        w = jnp.exp(mk - mk[0:1, :])
        w_ref[...] = w / jnp.sum(w, axis=0, keepdims=True)
        ids_ref[...] = idc
        sel = (key == -jnp.inf).astype(jnp.float32)
        cnt_ref[...] += jnp.sum(sel, axis=1)


_tc_router = pl.pallas_call(
    _tc_router_body,
    grid=(NT // TB,),
    in_specs=[pl.BlockSpec((E, TB), lambda i: (0, i))],
    out_specs=[
        pl.BlockSpec((E, TB), lambda i: (0, i)),
        pl.BlockSpec((K, TB), lambda i: (0, i)),
        pl.BlockSpec((K, TB), lambda i: (0, i)),
        pl.BlockSpec((E,), lambda i: (0,)),
    ],
    out_shape=(
        jax.ShapeDtypeStruct((E, NT), jnp.float32),
        jax.ShapeDtypeStruct((K, NT), jnp.float32),
        jax.ShapeDtypeStruct((K, NT), jnp.int32),
        jax.ShapeDtypeStruct((E,), jnp.float32),
    ),
)


def _hist_reduce_body(p_ref, t_ref, o_ref):
    o_ref[...] = jnp.sum(p_ref[...], axis=0) + t_ref[...]


def kernel(logits):
    xT_out, wT_tc, idsT_tc, cnt_tc = _tc_router(logits.T)
    w_sc, ids_sc, partials = _sc_router(logits[NT_TC:, :].reshape(-1))
    tokens_per_expert = pl.pallas_call(
        _hist_reduce_body,
        out_shape=jax.ShapeDtypeStruct((E,), jnp.float32),
    )(partials.reshape(NW, E), cnt_tc)
    wT = lax.dynamic_update_slice(wT_tc, w_sc.reshape(NT_SC, K).T, (0, NT_TC))
    idsT = lax.dynamic_update_slice(idsT_tc, ids_sc.reshape(NT_SC, K).T, (0, NT_TC))
    return (xT_out.T, wT.T, idsT.T, tokens_per_expert)
